# baseline (speedup 1.0000x reference)
import functools

import jax
import jax.numpy as jnp
from jax import lax
from jax.experimental import pallas as pl
from jax.experimental.pallas import tpu as pltpu
from jax.experimental.pallas import tpu_sc as plsc

B = 16384
D = 64
NC = 2
NS = 16
NW = NC * NS
L = 16
BPW = B // NW
C = 128
NCHUNK = BPW // C
NG = C // L


def _fast_sqrt(x):
    xm = jnp.maximum(x, jnp.float32(1e-30))
    i = plsc.bitcast(xm, jnp.int32)
    i = jnp.int32(0x5F3759DF) - (i >> 1)
    y = plsc.bitcast(i, jnp.float32)
    for _ in range(3):
        y = y * (jnp.float32(1.5) - jnp.float32(0.5) * xm * y * y)
    return xm * y


def _make_sc_call():
    mesh = plsc.VectorSubcoreMesh(core_axis_name="c", subcore_axis_name="s")

    @functools.partial(
        pl.kernel,
        out_type=jax.ShapeDtypeStruct((B,), jnp.float32),
        mesh=mesh,
        compiler_params=pltpu.CompilerParams(
            needs_layout_passes=False, use_tc_tiling_on_sc=False),
        scratch_types=[
            pltpu.VMEM((NCHUNK, C), jnp.int32),
            pltpu.VMEM((NCHUNK, C), jnp.int32),
            pltpu.VMEM((NCHUNK, C), jnp.int32),
            pltpu.VMEM((C, D), jnp.float32),
            pltpu.VMEM((C, D), jnp.float32),
            pltpu.VMEM((C, D), jnp.float32),
            pltpu.VMEM((C, D), jnp.float32),
            pltpu.VMEM((C, D), jnp.float32),
            pltpu.VMEM((C, D), jnp.float32),
            pltpu.VMEM((C, D), jnp.float32),
            pltpu.VMEM((C, D), jnp.float32),
            pltpu.VMEM((BPW,), jnp.float32),
            pltpu.SemaphoreType.DMA,
            pltpu.SemaphoreType.DMA,
        ],
    )
    def trans_h(head_hbm, tail_hbm, rel_hbm, ent_hbm, rel_emb_hbm, nv_hbm,
                out_hbm, idx_h, idx_t, idx_r, rows_h0, rows_h1, rows_t0,
                rows_t1, rows_r0, rows_r1, rows_n0, rows_n1, score_v,
                sem0, sem1):
        wid = lax.axis_index("s") * NC + lax.axis_index("c")
        pltpu.sync_copy(head_hbm.at[wid], idx_h)
        pltpu.sync_copy(tail_hbm.at[wid], idx_t)
        pltpu.sync_copy(rel_hbm.at[wid], idx_r)

        sems = (sem0, sem1)
        bufs = ((rows_h0, rows_t0, rows_r0, rows_n0),
                (rows_h1, rows_t1, rows_r1, rows_n1))

        def fire(jj):
            bb = jj & 1
            s = sems[bb]
            rh, rt, rr, rn = bufs[bb]
            return [
                pltpu.async_copy(ent_hbm.at[idx_h.at[jj]], rh, s),
                pltpu.async_copy(ent_hbm.at[idx_t.at[jj]], rt, s),
                pltpu.async_copy(rel_emb_hbm.at[idx_r.at[jj]], rr, s),
                pltpu.async_copy(nv_hbm.at[idx_r.at[jj]], rn, s),
            ]

        def compute(jj):
            rh, rt, rr, rn = bufs[jj & 1]

            def group(g, _):
                rid = g * L + lax.iota(jnp.int32, L)

                def dim_body(dd, carry):
                    p, a, c, q = carry
                    col = jnp.full((L,), dd, jnp.int32)
                    h = plsc.load_gather(rh, [rid, col])
                    t = plsc.load_gather(rt, [rid, col])
                    r = plsc.load_gather(rr, [rid, col])
                    n = plsc.load_gather(rn, [rid, col])
                    u = h - t + r
                    return (p + u * u, a + u * n, c + n * r, q + n * n)

                z = jnp.zeros((L,), jnp.float32)
                p, a, c, q = lax.fori_loop(0, D, dim_body, (z, z, z, z))
                s = a - c
                sq = p - jnp.float32(2.0) * s * a + s * s * q
                score_v[pl.ds(jj * C + g * L, L)] = _fast_sqrt(sq)
                return 0

            lax.fori_loop(0, NG, group, 0)

        pending = fire(0)
        for jj in range(NCHUNK):
            for hdl in pending:
                hdl.wait()
            if jj + 1 < NCHUNK:
                pending = fire(jj + 1)
            compute(jj)

        pltpu.sync_copy(score_v, out_hbm.at[pl.ds(wid * BPW, BPW)])

    return trans_h


_sc_call = _make_sc_call()


@jax.jit
def kernel(head, relation, tail, entity_embedding, relation_embedding,
           normal_vector):
    head3 = head.reshape(NW, NCHUNK, C)
    tail3 = tail.reshape(NW, NCHUNK, C)
    rel3 = relation.reshape(NW, NCHUNK, C)
    return _sc_call(head3, tail3, rel3, entity_embedding,
                    relation_embedding, normal_vector)


# speedup vs baseline: 1.4344x; 1.4344x over previous
import functools

import jax
import jax.numpy as jnp
from jax import lax
from jax.experimental import pallas as pl
from jax.experimental.pallas import tpu as pltpu
from jax.experimental.pallas import tpu_sc as plsc

B = 16384
D = 64
NC = 2
NS = 16
NW = NC * NS
L = 16
BPW = B // NW
C = 16
NCHUNK = BPW // C


def _fast_sqrt(x):
    xm = jnp.maximum(x, jnp.float32(1e-30))
    i = plsc.bitcast(xm, jnp.int32)
    i = jnp.int32(0x5F3759DF) - (i >> 1)
    y = plsc.bitcast(i, jnp.float32)
    for _ in range(3):
        y = y * (jnp.float32(1.5) - jnp.float32(0.5) * xm * y * y)
    return xm * y


def _make_sc_call():
    mesh = plsc.VectorSubcoreMesh(core_axis_name="c", subcore_axis_name="s")

    @functools.partial(
        pl.kernel,
        out_type=jax.ShapeDtypeStruct((B,), jnp.float32),
        mesh=mesh,
        compiler_params=pltpu.CompilerParams(
            needs_layout_passes=False, use_tc_tiling_on_sc=True),
        scratch_types=[
            pltpu.VMEM((BPW,), jnp.int32),
            pltpu.VMEM((BPW,), jnp.int32),
            pltpu.VMEM((BPW,), jnp.int32),
            pltpu.VMEM((8 * C, D), jnp.float32),
            pltpu.VMEM((8 * C, D), jnp.float32),
            pltpu.VMEM((8 * C, D), jnp.float32),
            pltpu.VMEM((8 * C, D), jnp.float32),
            pltpu.VMEM((BPW,), jnp.float32),
            pltpu.SemaphoreType.DMA,
            pltpu.SemaphoreType.DMA,
        ],
    )
    def trans_h(head_hbm, tail_hbm, rel_hbm, ent_hbm, rel_emb_hbm, nv_hbm,
                out_hbm, idx_h, idx_t, idx_r, bh, bt, br, bn,
                score_v, sem, semm):
        wid = lax.axis_index("s") * NC + lax.axis_index("c")
        base = pl.multiple_of(wid * BPW, 128)

        cp1 = pltpu.async_copy(head_hbm.at[pl.ds(base, BPW)], idx_h, semm)
        cp2 = pltpu.async_copy(tail_hbm.at[pl.ds(base, BPW)], idx_t, semm)
        cp3 = pltpu.async_copy(rel_hbm.at[pl.ds(base, BPW)], idx_r, semm)
        cp1.wait()
        cp2.wait()
        cp3.wait()

        lane = lax.iota(jnp.int32, L)
        two = jnp.float32(2.0)

        def chunk(jj, _):
            eh = idx_h[pl.ds(jj * C, L)]
            et = idx_t[pl.ds(jj * C, L)]
            er = idx_r[pl.ds(jj * C, L)]
            for l in range(L):
                r0 = pl.multiple_of((eh[l] >> 3) * 8, 8)
                pltpu.async_copy(ent_hbm.at[pl.ds(r0, 8), :],
                                 bh.at[pl.ds(l * 8, 8), :], sem)
                r1 = pl.multiple_of((et[l] >> 3) * 8, 8)
                pltpu.async_copy(ent_hbm.at[pl.ds(r1, 8), :],
                                 bt.at[pl.ds(l * 8, 8), :], sem)
                r2 = pl.multiple_of((er[l] >> 3) * 8, 8)
                pltpu.async_copy(rel_emb_hbm.at[pl.ds(r2, 8), :],
                                 br.at[pl.ds(l * 8, 8), :], sem)
                pltpu.async_copy(nv_hbm.at[pl.ds(r2, 8), :],
                                 bn.at[pl.ds(l * 8, 8), :], sem)
            for buf in (bh, bt, br, bn):
                for l in range(L):
                    pltpu.make_async_copy(
                        ent_hbm.at[pl.ds(0, 8), :],
                        buf.at[pl.ds(l * 8, 8), :], sem).wait()

            sqv = jnp.zeros((L,), jnp.float32)
            for l in range(L):
                rh = l * 8 + (eh[l] & 7)
                rt = l * 8 + (et[l] & 7)
                rr = l * 8 + (er[l] & 7)
                pv = jnp.zeros((L,), jnp.float32)
                av = jnp.zeros((L,), jnp.float32)
                cv = jnp.zeros((L,), jnp.float32)
                qv = jnp.zeros((L,), jnp.float32)
                for kk in range(D // L):
                    sl = pl.ds(L * kk, L)
                    h = bh[rh, sl]
                    t = bt[rt, sl]
                    r = br[rr, sl]
                    n = bn[rr, sl]
                    u = h - t + r
                    pv = pv + u * u
                    av = av + u * n
                    cv = cv + n * r
                    qv = qv + n * n
                p = jnp.sum(pv)
                a = jnp.sum(av)
                c = jnp.sum(cv)
                q = jnp.sum(qv)
                s = a - c
                sq = p - two * s * a + s * s * q
                sqv = jnp.where(lane == l, sq, sqv)
            score_v[pl.ds(jj * C, L)] = sqv
            return 0

        lax.fori_loop(0, NCHUNK, chunk, 0)

        def sqrt_pass(g, _):
            score_v[pl.ds(g * L, L)] = _fast_sqrt(score_v[pl.ds(g * L, L)])
            return 0

        lax.fori_loop(0, BPW // L, sqrt_pass, 0)
        pltpu.sync_copy(score_v, out_hbm.at[pl.ds(base, BPW)])

    return trans_h


_sc_call = _make_sc_call()


@jax.jit
def kernel(head, relation, tail, entity_embedding, relation_embedding,
           normal_vector):
    return _sc_call(head, tail, relation, entity_embedding,
                    relation_embedding, normal_vector)
